# Initial kernel scaffold; baseline (speedup 1.0000x reference)
import jax
import jax.numpy as jnp
from jax.experimental import pallas as pl


def kernel(x, edge_index, w, W0, b0, W1, b1, Wp, bp, Wv, bv):
    raise NotImplementedError("write your pallas kernel here")


# speedup vs baseline: 3.8501x; 3.8501x over previous
import functools

import jax
import jax.numpy as jnp
from jax import lax
from jax.experimental import pallas as pl
from jax.experimental.pallas import tpu as pltpu
from jax.experimental.pallas import tpu_sc as plsc

_NC = 2
_NS = 16
_NW = _NC * _NS
_CH = 80
_D = 128


def _round_up(v, m):
    return (v + m - 1) // m * m


def _make_sc_deg(E, NPAD):
    EPW = E // _NW
    NCH = EPW // _CH
    NR = NPAD // _D
    RPT = NR // _NS

    mesh = plsc.VectorSubcoreMesh(
        core_axis_name="c", subcore_axis_name="s",
        num_cores=_NC, num_subcores=_NS)

    @functools.partial(
        pl.kernel,
        out_type=jax.ShapeDtypeStruct((_NC, NR, _D), jnp.float32),
        mesh=mesh,
        scratch_types=dict(
            acc=pltpu.VMEM_SHARED((NR, _D), jnp.float32),
            hist=pltpu.VMEM((NR, _D), jnp.float32),
            idx_v=pltpu.VMEM((_CH,), jnp.int32),
            iota_v=pltpu.VMEM((NR,), jnp.int32),
        ),
        compiler_params=pltpu.CompilerParams(needs_layout_passes=False),
    )
    def deg_kernel(src_hbm, zblk_hbm, out_hbm, acc, hist, idx_v, iota_v):
        c = lax.axis_index("c")
        s = lax.axis_index("s")
        pltpu.sync_copy(zblk_hbm.at[pl.ds(0, NR)], hist)
        for m in range(NR // 16):
            iota_v[pl.ds(m * 16, 16)] = (
                lax.iota(jnp.int32, 16) + jnp.int32(m * 16))

        @pl.when(s == 0)
        def _():
            pltpu.sync_copy(zblk_hbm.at[pl.ds(0, NR)], acc)

        plsc.subcore_barrier()
        wbase = (c * _NS + s) * EPW
        one16 = jnp.ones((16,), jnp.float32)

        def body(k, carry):
            eb = wbase + k * _CH
            pltpu.sync_copy(src_hbm.at[pl.ds(eb, _CH)], idx_v)
            for b in range(_CH // 16):
                vec = idx_v[pl.ds(b * 16, 16)]
                row = lax.shift_right_logical(vec, 7)
                col = lax.bitwise_and(vec, jnp.int32(_D - 1))
                plsc.addupdate_scatter(hist, [row, col], one16)
            return carry

        lax.fori_loop(0, NCH, body, 0)
        pltpu.sync_copy(hist, acc.at[iota_v], add=True)
        plsc.subcore_barrier()

        @pl.when(s < NR // 8)
        def _():
            pltpu.sync_copy(acc.at[pl.ds(s * 8, 8)],
                            out_hbm.at[c, pl.ds(s * 8, 8)])

    return deg_kernel


def _make_sc_edge(E, NT, NPAD):
    EPW = E // _NW
    NCH = EPW // _CH
    RPT = NPAD // _NS

    mesh = plsc.VectorSubcoreMesh(
        core_axis_name="c", subcore_axis_name="s",
        num_cores=_NC, num_subcores=_NS)

    @functools.partial(
        pl.kernel,
        out_type=jax.ShapeDtypeStruct((_NC, NPAD, _D), jnp.float32),
        mesh=mesh,
        scratch_types=dict(
            acc=pltpu.VMEM_SHARED((NPAD, _D), jnp.float32),
            srcv=pltpu.VMEM((_CH,), jnp.int32),
            dstv=pltpu.VMEM((_CH,), jnp.int32),
            wv=pltpu.VMEM((EPW,), jnp.float32),
            rows=pltpu.VMEM((_CH, _D), jnp.float32),
        ),
        compiler_params=pltpu.CompilerParams(needs_layout_passes=False),
    )
    def edge_kernel(tab_hbm, src_hbm, dst_hbm, w_hbm, zblk_hbm, out_hbm,
                    acc, srcv, dstv, wv, rows):
        c = lax.axis_index("c")
        s = lax.axis_index("s")
        wbase = (c * _NS + s) * EPW
        pltpu.sync_copy(w_hbm.at[pl.ds(wbase, EPW)], wv)
        pltpu.sync_copy(zblk_hbm, acc.at[pl.ds(s * RPT, RPT)])
        plsc.subcore_barrier()

        def body(k, carry):
            eb = wbase + k * _CH
            pltpu.sync_copy(src_hbm.at[pl.ds(eb, _CH)], srcv)
            pltpu.sync_copy(dst_hbm.at[pl.ds(eb, _CH)], dstv)
            pltpu.sync_copy(tab_hbm.at[srcv], rows)
            kbase = k * _CH
            for i in range(_CH):
                bw = plsc.load_gather(
                    wv, [jnp.full((16,), i, jnp.int32) + kbase])
                for j in range(_D // 16):
                    sl = pl.ds(j * 16, 16)
                    rows[i, sl] = rows[i, sl] * bw
            pltpu.sync_copy(rows, acc.at[dstv], add=True)
            return carry

        lax.fori_loop(0, NCH, body, 0)
        plsc.subcore_barrier()
        pltpu.sync_copy(acc.at[pl.ds(s * RPT, RPT)],
                        out_hbm.at[c, pl.ds(s * RPT, RPT)])

    return edge_kernel


def _tc_prescale_body(x_ref, deg_ref, o_ref):
    d = deg_ref[0, 0] + deg_ref[1, 0]
    norm = 1.0 / jnp.maximum(d, 1.0)
    o_ref[...] = x_ref[...] * norm[:, None]


def _tc_prescale(x, degp, blk):
    N = x.shape[0]
    grid = N // blk
    return pl.pallas_call(
        _tc_prescale_body,
        grid=(grid,),
        in_specs=[
            pl.BlockSpec((blk, _D), lambda i: (i, 0)),
            pl.BlockSpec((_NC, 1, blk), lambda i: (0, 0, i)),
        ],
        out_specs=pl.BlockSpec((blk, _D), lambda i: (i, 0)),
        out_shape=jax.ShapeDtypeStruct((N, _D), jnp.float32),
    )(x, degp)


def _tc_mid_body(agg_ref, W_ref, b_ref, deg_ref, o_ref):
    a = agg_ref[0] + agg_ref[1]
    h = jnp.dot(a, W_ref[...], preferred_element_type=jnp.float32)
    h = h + b_ref[...]
    h = jnp.maximum(h, 0.0)
    d = deg_ref[0, 0] + deg_ref[1, 0]
    norm = 1.0 / jnp.maximum(d, 1.0)
    o_ref[...] = h * norm[:, None]


def _tc_mid(agg, W, b2, degp, blk):
    NPAD = agg.shape[1]
    grid = NPAD // blk
    return pl.pallas_call(
        _tc_mid_body,
        grid=(grid,),
        in_specs=[
            pl.BlockSpec((_NC, blk, _D), lambda i: (0, i, 0)),
            pl.BlockSpec((_D, _D), lambda i: (0, 0)),
            pl.BlockSpec((1, _D), lambda i: (0, 0)),
            pl.BlockSpec((_NC, 1, blk), lambda i: (0, 0, i)),
        ],
        out_specs=pl.BlockSpec((blk, _D), lambda i: (i, 0)),
        out_shape=jax.ShapeDtypeStruct((NPAD, _D), jnp.float32),
    )(agg, W, b2, degp)


def _make_tc_final_body(ngrid, n_real):
    def body(agg_ref, W_ref, b_ref, Wp_ref, bp_ref, Wv_ref, bv_ref,
             pi_ref, v_ref, colsum):
        i = pl.program_id(0)
        a = agg_ref[0] + agg_ref[1]
        h2 = jnp.dot(a, W_ref[...], preferred_element_type=jnp.float32)
        h2 = h2 + b_ref[...]
        pi_ref[...] = (jnp.dot(h2, Wp_ref[...],
                               preferred_element_type=jnp.float32)
                       + bp_ref[...])
        blocksum = jnp.sum(a, axis=0, keepdims=True)

        @pl.when(i == 0)
        def _():
            colsum[...] = blocksum

        @pl.when(i > 0)
        def _():
            colsum[...] = colsum[...] + blocksum

        @pl.when(i == ngrid - 1)
        def _():
            m = colsum[...] / float(n_real)
            hv = jnp.dot(m, W_ref[...],
                         preferred_element_type=jnp.float32) + b_ref[...]
            v_ref[...] = (jnp.dot(hv, Wv_ref[...],
                                  preferred_element_type=jnp.float32)
                          + bv_ref[...])

    return body


def _tc_final(agg, W, b2, Wp, bp2, Wv, bv2, blk, n_real):
    NPAD = agg.shape[1]
    grid = NPAD // blk
    return pl.pallas_call(
        _make_tc_final_body(grid, n_real),
        grid=(grid,),
        in_specs=[
            pl.BlockSpec((_NC, blk, _D), lambda i: (0, i, 0)),
            pl.BlockSpec((_D, _D), lambda i: (0, 0)),
            pl.BlockSpec((1, _D), lambda i: (0, 0)),
            pl.BlockSpec((_D, 1), lambda i: (0, 0)),
            pl.BlockSpec((1, 1), lambda i: (0, 0)),
            pl.BlockSpec((_D, 1), lambda i: (0, 0)),
            pl.BlockSpec((1, 1), lambda i: (0, 0)),
        ],
        out_specs=[
            pl.BlockSpec((blk, 1), lambda i: (i, 0)),
            pl.BlockSpec((1, 1), lambda i: (0, 0)),
        ],
        out_shape=[
            jax.ShapeDtypeStruct((NPAD, 1), jnp.float32),
            jax.ShapeDtypeStruct((1, 1), jnp.float32),
        ],
        scratch_shapes=[pltpu.VMEM((1, _D), jnp.float32)],
    )(agg, W, b2, Wp, bp2, Wv, bv2)


def kernel(x, edge_index, w, W0, b0, W1, b1, Wp, bp, Wv, bv):
    N, D = x.shape
    E = edge_index.shape[1]
    NPAD = _round_up(N, _NS * 16)
    RPT = NPAD // _NS

    src = edge_index[0]
    dst = edge_index[1]

    zblk = jnp.zeros((RPT, _D), jnp.float32)

    degp = _make_sc_deg(E, NPAD)(src, zblk)
    degr = degp.reshape(_NC, 1, NPAD)

    x_p = jnp.pad(x, ((0, NPAD - N), (0, 0)))
    hn0 = _tc_prescale(x_p, degr, 512)
    agg0 = _make_sc_edge(E, NPAD, NPAD)(hn0, src, dst, w, zblk)
    h1n = _tc_mid(agg0, W0, b0.reshape(1, _D), degr, 512)
    agg1 = _make_sc_edge(E, NPAD, NPAD)(h1n, src, dst, w, zblk)
    PI_p, V = _tc_final(agg1, W1, b1.reshape(1, _D),
                        Wp, bp.reshape(1, 1), Wv, bv.reshape(1, 1),
                        512, N)
    return (PI_p[:N], V)
